# baseline (speedup 1.0000x reference)
import functools

import jax
import jax.numpy as jnp
from jax import lax
from jax.experimental import pallas as pl
from jax.experimental.pallas import tpu as pltpu
from jax.experimental.pallas import tpu_sc as plsc

VOCAB = 1000000
D_MODEL = 64
BATCH = 4096
SEQ = 200

_NW = 32
_RPW = BATCH // _NW

_NBLK = VOCAB // 128
_TAIL = VOCAB - _NBLK * 128
_BLK_PER_W = _NBLK // _NW


@functools.cache
def _build_k1():
    mesh = plsc.VectorSubcoreMesh(core_axis_name="c", subcore_axis_name="s")

    @functools.partial(
        pl.kernel,
        mesh=mesh,
        compiler_params=pltpu.CompilerParams(needs_layout_passes=False),
        out_type=jax.ShapeDtypeStruct((VOCAB // 2, 128), jnp.float32),
        scratch_types=[
            pltpu.VMEM((D_MODEL, 128), jnp.float32),
            pltpu.VMEM((D_MODEL, 128), jnp.float32),
        ],
    )
    def _k1(tt_hbm, tail_hbm, tp_hbm, vb, wb):
        wid = lax.axis_index("s") * 2 + lax.axis_index("c")
        iota16 = lax.iota(jnp.int32, 16)
        rows = (iota16, iota16 + 16, iota16 + 32, iota16 + 48)

        def transpose_cols():
            def col_body(c, carry):
                p = c // 2
                h = c - 2 * p
                for k in range(4):
                    vals = plsc.load_gather(
                        vb, [rows[k], jnp.full((16,), c, jnp.int32)]
                    )
                    wb[p, pl.ds(64 * h + 16 * k, 16)] = vals
                return carry

            lax.fori_loop(0, 128, col_body, 0)

        def do_block(col0):
            col0 = pl.multiple_of(col0, 128)
            pltpu.sync_copy(tt_hbm.at[:, pl.ds(col0, 128)], vb)
            transpose_cols()
            prow = pl.multiple_of(col0 // 2, 8)
            pltpu.sync_copy(wb, tp_hbm.at[pl.ds(prow, 64)])

        def blk_body(j, carry):
            do_block((wid + _NW * j) * 128)
            return carry

        lax.fori_loop(0, _BLK_PER_W, blk_body, 0)

        @pl.when(wid < _NBLK - _NW * _BLK_PER_W)
        def _():
            do_block((_NW * _BLK_PER_W + wid) * 128)

        @pl.when(wid == 4)
        def _():
            pltpu.sync_copy(tail_hbm, wb.at[pl.ds(0, _TAIL // 2)])
            pltpu.sync_copy(
                wb.at[pl.ds(0, _TAIL // 2)],
                tp_hbm.at[pl.ds(VOCAB // 2 - _TAIL // 2, _TAIL // 2)],
            )

    return _k1


@functools.cache
def _build_k2():
    mesh = plsc.VectorSubcoreMesh(core_axis_name="c", subcore_axis_name="s")

    @functools.partial(
        pl.kernel,
        mesh=mesh,
        compiler_params=pltpu.CompilerParams(use_tc_tiling_on_sc=False),
        out_type=jax.ShapeDtypeStruct((BATCH, SEQ, D_MODEL), jnp.float32),
        scratch_types=[
            pltpu.VMEM((_RPW, SEQ), jnp.int32),
            pltpu.VMEM((SEQ, D_MODEL), jnp.float32),
            pltpu.VMEM((SEQ, D_MODEL), jnp.float32),
            pltpu.SemaphoreType.DMA,
            pltpu.SemaphoreType.DMA,
        ],
    )
    def _k2(x_hbm, table_hbm, out_hbm, idx_v, buf0, buf1, sem0, sem1):
        wid = lax.axis_index("s") * 2 + lax.axis_index("c")
        row_base = wid * _RPW
        pltpu.sync_copy(x_hbm.at[pl.ds(row_base, _RPW)], idx_v)

        def gather(r, buf, sem):
            return pltpu.async_copy(table_hbm.at[idx_v.at[r]], buf, sem)

        def gwait(buf, sem):
            pltpu.make_async_copy(table_hbm.at[idx_v.at[0]], buf, sem).wait()

        def put(r, buf):
            pltpu.sync_copy(buf, out_hbm.at[row_base + r])

        gather(0, buf0, sem0)

        def body(i, carry):
            r = 2 * i
            gather(r + 1, buf1, sem1)
            gwait(buf0, sem0)
            put(r, buf0)
            gather(r + 2, buf0, sem0)
            gwait(buf1, sem1)
            put(r + 1, buf1)
            return carry

        lax.fori_loop(0, _RPW // 2 - 1, body, 0)

        r = _RPW - 2
        gather(r + 1, buf1, sem1)
        gwait(buf0, sem0)
        put(r, buf0)
        gwait(buf1, sem1)
        put(r + 1, buf1)

    return _k2


def kernel(x, table):
    tail = table[VOCAB - _TAIL :].reshape(_TAIL // 2, 128)
    tpairs = _build_k1()(table.T, tail)
    t_lin = tpairs.reshape(VOCAB, D_MODEL)
    return _build_k2()(x, t_lin)


# speedup vs baseline: 1.6464x; 1.6464x over previous
import functools

import jax
import jax.numpy as jnp
from jax import lax
from jax.experimental import pallas as pl
from jax.experimental.pallas import tpu as pltpu
from jax.experimental.pallas import tpu_sc as plsc

VOCAB = 1000000
D_MODEL = 64
BATCH = 4096
SEQ = 200

_NW = 32
_RPW = BATCH // _NW


@functools.cache
def _build_sc_gather(seq):
    mesh = plsc.VectorSubcoreMesh(core_axis_name="c", subcore_axis_name="s")

    @functools.partial(
        pl.kernel,
        mesh=mesh,
        compiler_params=pltpu.CompilerParams(use_tc_tiling_on_sc=False),
        out_type=jax.ShapeDtypeStruct((BATCH, seq, D_MODEL), jnp.float32),
        scratch_types=[
            pltpu.VMEM((_RPW, seq), jnp.int32),
            pltpu.VMEM((seq, D_MODEL), jnp.float32),
            pltpu.VMEM((seq, D_MODEL), jnp.float32),
            pltpu.SemaphoreType.DMA,
            pltpu.SemaphoreType.DMA,
        ],
    )
    def _sc_gather(x_hbm, table_hbm, out_hbm, idx_v, buf0, buf1, sem0, sem1):
        wid = lax.axis_index("s") * 2 + lax.axis_index("c")
        row_base = wid * _RPW
        pltpu.sync_copy(x_hbm.at[pl.ds(row_base, _RPW)], idx_v)

        def gather(r, buf, sem):
            return pltpu.async_copy(table_hbm.at[idx_v.at[r]], buf, sem)

        def gwait(buf, sem):
            pltpu.make_async_copy(table_hbm.at[idx_v.at[0]], buf, sem).wait()

        def put(r, buf):
            pltpu.sync_copy(buf, out_hbm.at[row_base + r])

        gather(0, buf0, sem0)

        def body(i, carry):
            r = 2 * i
            gather(r + 1, buf1, sem1)
            gwait(buf0, sem0)
            put(r, buf0)
            gather(r + 2, buf0, sem0)
            gwait(buf1, sem1)
            put(r + 1, buf1)
            return carry

        lax.fori_loop(0, _RPW // 2 - 1, body, 0)

        r = _RPW - 2
        gather(r + 1, buf1, sem1)
        gwait(buf0, sem0)
        put(r, buf0)
        gwait(buf1, sem1)
        put(r + 1, buf1)

    return _sc_gather


def kernel(x, table):
    h = SEQ // 2
    g = _build_sc_gather(h)
    o1 = g(x[:, :h], table)
    o2 = g(x[:, h:], table)
    return jnp.concatenate([o1, o2], axis=1)


# speedup vs baseline: 2.0049x; 1.2177x over previous
import functools

import jax
import jax.numpy as jnp
from jax import lax
from jax.experimental import pallas as pl
from jax.experimental.pallas import tpu as pltpu
from jax.experimental.pallas import tpu_sc as plsc

VOCAB = 1000000
D_MODEL = 64
BATCH = 4096
SEQ = 200

_NW = 32
_RPW = BATCH // _NW


@functools.cache
def _build_sc_gather():
    mesh = plsc.VectorSubcoreMesh(core_axis_name="c", subcore_axis_name="s")

    @functools.partial(
        pl.kernel,
        mesh=mesh,
        compiler_params=pltpu.CompilerParams(use_tc_tiling_on_sc=False),
        out_type=jax.ShapeDtypeStruct((BATCH, SEQ, D_MODEL), jnp.float32),
        scratch_types=[
            pltpu.VMEM((_RPW, SEQ), jnp.int32),
            pltpu.VMEM((SEQ, D_MODEL), jnp.float32),
            pltpu.VMEM((SEQ, D_MODEL), jnp.float32),
            pltpu.SemaphoreType.DMA,
            pltpu.SemaphoreType.DMA,
        ],
    )
    def _sc_gather(x_hbm, table_hbm, out_hbm, idx_v, buf0, buf1, sem0, sem1):
        wid = lax.axis_index("s") * 2 + lax.axis_index("c")
        row_base = wid * _RPW
        pltpu.sync_copy(x_hbm.at[pl.ds(row_base, _RPW)], idx_v)

        def gather(r, buf, sem):
            return pltpu.async_copy(table_hbm.at[idx_v.at[r]], buf, sem)

        def gwait(buf, sem):
            pltpu.make_async_copy(table_hbm.at[idx_v.at[0]], buf, sem).wait()

        def put(r, buf):
            pltpu.sync_copy(buf, out_hbm.at[row_base + r])

        gather(0, buf0, sem0)

        def body(i, carry):
            r = 2 * i
            gather(r + 1, buf1, sem1)
            gwait(buf0, sem0)
            put(r, buf0)
            gather(r + 2, buf0, sem0)
            gwait(buf1, sem1)
            put(r + 1, buf1)
            return carry

        lax.fori_loop(0, _RPW // 2 - 1, body, 0)

        r = _RPW - 2
        gather(r + 1, buf1, sem1)
        gwait(buf0, sem0)
        put(r, buf0)
        gwait(buf1, sem1)
        put(r + 1, buf1)

    return _sc_gather


def kernel(x, table):
    return _build_sc_gather()(x, table)
